# Initial kernel scaffold; baseline (speedup 1.0000x reference)
import jax
import jax.numpy as jnp
from jax.experimental import pallas as pl


def kernel(x, edge_index, edge_attr, q, tclass, batch, Wx, bx, We, be, W1, b1, W2, b2, Wq1, bq1, Wq2, bq2, Temb, Wy1, by1, Wy2, by2, Wa1, ba1, Wa2, ba2):
    raise NotImplementedError("write your pallas kernel here")


# speedup vs baseline: 3.0157x; 3.0157x over previous
import functools

import jax
import jax.numpy as jnp
from jax import lax
from jax.experimental import pallas as pl
from jax.experimental.pallas import tpu as pltpu
from jax.experimental.pallas import tpu_sc as plsc

N = 10000
E = 320000
XD = 128
ED = 16
H = 128
B = 64
QD = 6
TV = 8
L = 3

NC = 2
NS = 16
NW = NC * NS
EPT = E // NW
K = 80
CHUNKS = EPT // K
NP = 10240
RPT = NP // NS
ZR = 128
NZ = RPT // ZR

LANES = 16


def _sc_body(h_hbm, ep_hbm, src_hbm, dst_hbm, out_hbm,
             sidx, didx, rows, epv, zbuf, aggr_sh, sem):
    cid = lax.axis_index("c")
    sid = lax.axis_index("s")
    wid = cid * NS + sid

    zeros16 = jnp.zeros((LANES,), jnp.float32)

    def zrow(r, carry):
        for j in range(H // LANES):
            zbuf[r, pl.ds(j * LANES, LANES)] = zeros16
        return carry

    lax.fori_loop(0, ZR, zrow, 0)

    row0 = sid * RPT

    def zcopy(i, carry):
        pltpu.sync_copy(zbuf, aggr_sh.at[pl.ds(row0 + i * ZR, ZR)])
        return carry

    lax.fori_loop(0, NZ, zcopy, 0)
    plsc.subcore_barrier()

    e0 = wid * EPT

    def chunk(c, carry):
        base = e0 + c * K
        pltpu.sync_copy(src_hbm.at[pl.ds(base, K)], sidx)
        pltpu.sync_copy(dst_hbm.at[pl.ds(base, K)], didx)
        gat = pltpu.async_copy(h_hbm.at[sidx], rows, sem)
        pltpu.sync_copy(ep_hbm.at[pl.ds(base, K)], epv)
        gat.wait()

        def rowfn(r, c2):
            for j in range(H // LANES):
                sl = pl.ds(j * LANES, LANES)
                rows[r, sl] = jnp.maximum(rows[r, sl] + epv[r, sl], 0.0)
            return c2

        lax.fori_loop(0, K, rowfn, 0)
        pltpu.sync_copy(rows, aggr_sh.at[didx], add=True)
        return carry

    lax.fori_loop(0, CHUNKS, chunk, 0)
    plsc.subcore_barrier()

    def wcopy(i, carry):
        sl = pl.ds(row0 + i * ZR, ZR)
        pltpu.sync_copy(aggr_sh.at[sl], out_hbm.at[cid, sl])
        return carry

    lax.fori_loop(0, NZ, wcopy, 0)


@jax.jit
def _sc_aggr(h, ep, src, dst):
    mesh = plsc.VectorSubcoreMesh(
        core_axis_name="c", subcore_axis_name="s",
        num_cores=NC, num_subcores=NS)
    fn = pl.kernel(
        _sc_body,
        out_type=jax.ShapeDtypeStruct((NC, NP, H), jnp.float32),
        mesh=mesh,
        scratch_types=[
            pltpu.VMEM((K,), jnp.int32),
            pltpu.VMEM((K,), jnp.int32),
            pltpu.VMEM((K, H), jnp.float32),
            pltpu.VMEM((K, H), jnp.float32),
            pltpu.VMEM((ZR, H), jnp.float32),
            pltpu.VMEM_SHARED((NP, H), jnp.float32),
            pltpu.SemaphoreType.DMA,
        ],
        name="gine_sc_aggr",
    )
    return fn(h, ep, src, dst)


_MBLK = 2000


def _h0_body(x_ref, w_ref, b_ref, o_ref):
    o_ref[...] = jnp.dot(x_ref[...], w_ref[...],
                         preferred_element_type=jnp.float32) + b_ref[...]


def _tc_h0(x, Wx, bx):
    return pl.pallas_call(
        _h0_body,
        grid=(N // _MBLK,),
        in_specs=[
            pl.BlockSpec((_MBLK, XD), lambda i: (i, 0)),
            pl.BlockSpec((XD, H), lambda i: (0, 0)),
            pl.BlockSpec((1, H), lambda i: (0, 0)),
        ],
        out_specs=pl.BlockSpec((_MBLK, H), lambda i: (i, 0)),
        out_shape=jax.ShapeDtypeStruct((N, H), jnp.float32),
    )(x, Wx, bx.reshape(1, H))


_EBLK = 4000


def _ep_body(ea_ref, w_ref, b_ref, o_ref):
    o_ref[...] = jnp.dot(ea_ref[...], w_ref[...],
                         preferred_element_type=jnp.float32) + b_ref[...]


def _tc_ep(edge_attr, We_l, be_l):
    return pl.pallas_call(
        _ep_body,
        grid=(E // _EBLK,),
        in_specs=[
            pl.BlockSpec((_EBLK, ED), lambda i: (i, 0)),
            pl.BlockSpec((ED, H), lambda i: (0, 0)),
            pl.BlockSpec((1, H), lambda i: (0, 0)),
        ],
        out_specs=pl.BlockSpec((_EBLK, H), lambda i: (i, 0)),
        out_shape=jax.ShapeDtypeStruct((E, H), jnp.float32),
    )(edge_attr, We_l, be_l.reshape(1, H))


def _mlp_body(h_ref, a_ref, w1_ref, b1_ref, w2_ref, b2_ref, o_ref):
    hin = h_ref[...] + a_ref[0] + a_ref[1]
    t = jnp.maximum(
        jnp.dot(hin, w1_ref[...], preferred_element_type=jnp.float32)
        + b1_ref[...], 0.0)
    o_ref[...] = jnp.maximum(
        jnp.dot(t, w2_ref[...], preferred_element_type=jnp.float32)
        + b2_ref[...], 0.0)


def _tc_mlp(h, aggr2, W1_l, b1_l, W2_l, b2_l):
    return pl.pallas_call(
        _mlp_body,
        grid=(N // _MBLK,),
        in_specs=[
            pl.BlockSpec((_MBLK, H), lambda i: (i, 0)),
            pl.BlockSpec((NC, _MBLK, H), lambda i: (0, i, 0)),
            pl.BlockSpec((H, H), lambda i: (0, 0)),
            pl.BlockSpec((1, H), lambda i: (0, 0)),
            pl.BlockSpec((H, H), lambda i: (0, 0)),
            pl.BlockSpec((1, H), lambda i: (0, 0)),
        ],
        out_specs=pl.BlockSpec((_MBLK, H), lambda i: (i, 0)),
        out_shape=jax.ShapeDtypeStruct((N, H), jnp.float32),
    )(h, aggr2, W1_l, b1_l.reshape(1, H), W2_l, b2_l.reshape(1, H))


def _head_body(h_ref, batch_ref, q_ref, tc_ref, temb_ref,
               wq1_ref, bq1_ref, wq2_ref, bq2_ref,
               wy1g_ref, wy1q_ref, wy1t_ref, by1_ref, wy2_ref, by2_ref,
               wa1g_ref, wa1q_ref, wa1t_ref, ba1_ref, wa2_ref, ba2_ref,
               y_ref, ax_ref):
    f32 = jnp.float32
    oh = (lax.broadcasted_iota(jnp.int32, (N, B), 1)
          == batch_ref[...]).astype(f32)
    h = h_ref[...]
    g = lax.dot_general(oh, h, (((0,), (0,)), ((), ())),
                        preferred_element_type=f32)
    ones = jnp.ones((N, 1), f32)
    cnt = lax.dot_general(oh, ones, (((0,), (0,)), ((), ())),
                          preferred_element_type=f32)
    g = g / jnp.maximum(cnt, 1.0)

    qh = jnp.maximum(
        jnp.dot(q_ref[...], wq1_ref[...], preferred_element_type=f32)
        + bq1_ref[...], 0.0)
    qh = jnp.dot(qh, wq2_ref[...], preferred_element_type=f32) + bq2_ref[...]

    oht = (lax.broadcasted_iota(jnp.int32, (B, TV), 1)
           == tc_ref[...]).astype(f32)
    th = jnp.dot(oht, temb_ref[...], preferred_element_type=f32)

    zy = jnp.maximum(
        jnp.dot(g, wy1g_ref[...], preferred_element_type=f32)
        + jnp.dot(qh, wy1q_ref[...], preferred_element_type=f32)
        + jnp.dot(th, wy1t_ref[...], preferred_element_type=f32)
        + by1_ref[...], 0.0)
    y_ref[...] = jnp.dot(zy, wy2_ref[...], preferred_element_type=f32) \
        + by2_ref[...]

    za = jnp.maximum(
        jnp.dot(g, wa1g_ref[...], preferred_element_type=f32)
        + jnp.dot(qh, wa1q_ref[...], preferred_element_type=f32)
        + jnp.dot(th, wa1t_ref[...], preferred_element_type=f32)
        + ba1_ref[...], 0.0)
    ax_ref[...] = jnp.dot(za, wa2_ref[...], preferred_element_type=f32) \
        + ba2_ref[...]


def _tc_head(h, batch, q, tclass, Temb,
             Wq1, bq1, Wq2, bq2, Wy1, by1, Wy2, by2, Wa1, ba1, Wa2, ba2):
    y, ax = pl.pallas_call(
        _head_body,
        out_shape=(
            jax.ShapeDtypeStruct((B, 1), jnp.float32),
            jax.ShapeDtypeStruct((B, 6), jnp.float32),
        ),
    )(h, batch.reshape(N, 1), q, tclass.reshape(B, 1), Temb,
      Wq1, bq1.reshape(1, H), Wq2, bq2.reshape(1, H),
      Wy1[:H], Wy1[H:2 * H], Wy1[2 * H:], by1.reshape(1, H),
      Wy2, by2.reshape(1, 1),
      Wa1[:H], Wa1[H:2 * H], Wa1[2 * H:], ba1.reshape(1, H),
      Wa2, ba2.reshape(1, 6))
    return y.reshape(-1), ax


def kernel(x, edge_index, edge_attr, q, tclass, batch,
           Wx, bx, We, be, W1, b1, W2, b2, Wq1, bq1, Wq2, bq2, Temb,
           Wy1, by1, Wy2, by2, Wa1, ba1, Wa2, ba2):
    src = edge_index[0]
    dst = edge_index[1]
    h = _tc_h0(x, Wx, bx)
    for l in range(L):
        ep = _tc_ep(edge_attr, We[l], be[l])
        aggr2 = _sc_aggr(h, ep, src, dst)
        h = _tc_mlp(h, aggr2, W1[l], b1[l], W2[l], b2[l])
    return _tc_head(h, batch, q, tclass, Temb,
                    Wq1, bq1, Wq2, bq2, Wy1, by1, Wy2, by2,
                    Wa1, ba1, Wa2, ba2)


# speedup vs baseline: 4.8175x; 1.5975x over previous
import functools

import jax
import jax.numpy as jnp
from jax import lax
from jax.experimental import pallas as pl
from jax.experimental.pallas import tpu as pltpu
from jax.experimental.pallas import tpu_sc as plsc

N = 10000
E = 320000
XD = 128
ED = 16
H = 128
B = 64
QD = 6
TV = 8
L = 3

NC = 2
NS = 16
NW = NC * NS
EPT = E // NW
K = 40
CHUNKS = EPT // K
RING = 2
T2 = CHUNKS // RING
NP = 10240
RPT = NP // NS
ZR = 128
NZ = RPT // ZR

LANES = 16


def _sc_body(h_hbm, ep_hbm, src_hbm, dst_hbm, out_hbm,
             sidx_all, rows, epv, msg, didx, aggr_sh,
             sem_g, sem_e, sem_s, sem_d):
    cid = lax.axis_index("c")
    sid = lax.axis_index("s")
    wid = cid * NS + sid
    e0 = wid * EPT
    row0 = sid * RPT

    zeros16 = jnp.zeros((LANES,), jnp.float32)

    def zrow(r, carry):
        for j in range(H // LANES):
            msg[0][r, pl.ds(j * LANES, LANES)] = zeros16
        return carry

    lax.fori_loop(0, K, zrow, 0)

    def zcopy(i, carry):
        pltpu.sync_copy(msg[0], aggr_sh.at[pl.ds(row0 + i * K, K)])
        return carry

    lax.fori_loop(0, RPT // K, zcopy, 0)

    pltpu.sync_copy(src_hbm.at[pl.ds(e0, EPT)], sidx_all)
    plsc.subcore_barrier()

    def issue_ge(c, b):
        base = e0 + c * K
        pltpu.async_copy(h_hbm.at[sidx_all.at[pl.ds(c * K, K)]],
                         rows[b], sem_g.at[b])
        pltpu.async_copy(ep_hbm.at[pl.ds(base, K)], epv[b], sem_e.at[b])

    def wait_ge(b):
        pltpu.make_async_copy(
            h_hbm.at[sidx_all.at[pl.ds(0, K)]], rows[b], sem_g.at[b]).wait()
        pltpu.make_async_copy(
            ep_hbm.at[pl.ds(0, K)], epv[b], sem_e.at[b]).wait()

    def wait_scatter(b):
        pltpu.make_async_copy(
            msg[b], aggr_sh.at[didx[b]], sem_s.at[b]).wait()

    for b in range(RING):
        issue_ge(b, b)

    def step(t2, carry):
        for b in range(RING):
            c = RING * t2 + b
            wait_ge(b)
            @pl.when(t2 > 0)
            def _():
                wait_scatter(b)
            pltpu.async_copy(dst_hbm.at[pl.ds(e0 + c * K, K)],
                             didx[b], sem_d.at[b])
            rk, ek, mk = rows[b], epv[b], msg[b]

            def rowfn(r, c2):
                for j in range(H // LANES):
                    sl = pl.ds(j * LANES, LANES)
                    mk[r, sl] = jnp.maximum(rk[r, sl] + ek[r, sl], 0.0)
                return c2

            lax.fori_loop(0, K, rowfn, 0)
            pltpu.make_async_copy(
                dst_hbm.at[pl.ds(0, K)], didx[b], sem_d.at[b]).wait()
            pltpu.async_copy(msg[b], aggr_sh.at[didx[b]],
                             sem_s.at[b], add=True)

            @pl.when(t2 < T2 - 1)
            def _():
                issue_ge(c + RING, b)
        return carry

    lax.fori_loop(0, T2, step, 0)

    for b in range(RING):
        wait_scatter(b)
    plsc.subcore_barrier()

    def wcopy(i, carry):
        sl = pl.ds(row0 + i * ZR, ZR)
        pltpu.sync_copy(aggr_sh.at[sl], out_hbm.at[cid, sl])
        return carry

    lax.fori_loop(0, NZ, wcopy, 0)


@jax.jit
def _sc_aggr(h, ep, src, dst):
    mesh = plsc.VectorSubcoreMesh(
        core_axis_name="c", subcore_axis_name="s",
        num_cores=NC, num_subcores=NS)
    fn = pl.kernel(
        _sc_body,
        out_type=jax.ShapeDtypeStruct((NC, NP, H), jnp.float32),
        mesh=mesh,
        scratch_types=[
            pltpu.VMEM((EPT,), jnp.int32),
            [pltpu.VMEM((K, H), jnp.float32)] * RING,
            [pltpu.VMEM((K, H), jnp.float32)] * RING,
            [pltpu.VMEM((K, H), jnp.float32)] * RING,
            [pltpu.VMEM((K,), jnp.int32)] * RING,
            pltpu.VMEM_SHARED((NP, H), jnp.float32),
            pltpu.SemaphoreType.DMA((RING,)),
            pltpu.SemaphoreType.DMA((RING,)),
            pltpu.SemaphoreType.DMA((RING,)),
            pltpu.SemaphoreType.DMA((RING,)),
        ],
        name="gine_sc_aggr",
    )
    return fn(h, ep, src, dst)


_MBLK = 2000


def _h0_body(x_ref, w_ref, b_ref, o_ref):
    o_ref[...] = jnp.dot(x_ref[...], w_ref[...],
                         preferred_element_type=jnp.float32) + b_ref[...]


def _tc_h0(x, Wx, bx):
    return pl.pallas_call(
        _h0_body,
        grid=(N // _MBLK,),
        in_specs=[
            pl.BlockSpec((_MBLK, XD), lambda i: (i, 0)),
            pl.BlockSpec((XD, H), lambda i: (0, 0)),
            pl.BlockSpec((1, H), lambda i: (0, 0)),
        ],
        out_specs=pl.BlockSpec((_MBLK, H), lambda i: (i, 0)),
        out_shape=jax.ShapeDtypeStruct((N, H), jnp.float32),
    )(x, Wx, bx.reshape(1, H))


_EBLK = 4000


def _ep_body(ea_ref, w_ref, b_ref, o_ref):
    o_ref[...] = jnp.dot(ea_ref[...], w_ref[...],
                         preferred_element_type=jnp.float32) + b_ref[...]


def _tc_ep(edge_attr, We_l, be_l):
    return pl.pallas_call(
        _ep_body,
        grid=(E // _EBLK,),
        in_specs=[
            pl.BlockSpec((_EBLK, ED), lambda i: (i, 0)),
            pl.BlockSpec((ED, H), lambda i: (0, 0)),
            pl.BlockSpec((1, H), lambda i: (0, 0)),
        ],
        out_specs=pl.BlockSpec((_EBLK, H), lambda i: (i, 0)),
        out_shape=jax.ShapeDtypeStruct((E, H), jnp.float32),
    )(edge_attr, We_l, be_l.reshape(1, H))


def _mlp_body(h_ref, a_ref, w1_ref, b1_ref, w2_ref, b2_ref, o_ref):
    hin = h_ref[...] + a_ref[0] + a_ref[1]
    t = jnp.maximum(
        jnp.dot(hin, w1_ref[...], preferred_element_type=jnp.float32)
        + b1_ref[...], 0.0)
    o_ref[...] = jnp.maximum(
        jnp.dot(t, w2_ref[...], preferred_element_type=jnp.float32)
        + b2_ref[...], 0.0)


def _tc_mlp(h, aggr2, W1_l, b1_l, W2_l, b2_l):
    return pl.pallas_call(
        _mlp_body,
        grid=(N // _MBLK,),
        in_specs=[
            pl.BlockSpec((_MBLK, H), lambda i: (i, 0)),
            pl.BlockSpec((NC, _MBLK, H), lambda i: (0, i, 0)),
            pl.BlockSpec((H, H), lambda i: (0, 0)),
            pl.BlockSpec((1, H), lambda i: (0, 0)),
            pl.BlockSpec((H, H), lambda i: (0, 0)),
            pl.BlockSpec((1, H), lambda i: (0, 0)),
        ],
        out_specs=pl.BlockSpec((_MBLK, H), lambda i: (i, 0)),
        out_shape=jax.ShapeDtypeStruct((N, H), jnp.float32),
    )(h, aggr2, W1_l, b1_l.reshape(1, H), W2_l, b2_l.reshape(1, H))


def _head_body(h_ref, batch_ref, q_ref, tc_ref, temb_ref,
               wq1_ref, bq1_ref, wq2_ref, bq2_ref,
               wy1g_ref, wy1q_ref, wy1t_ref, by1_ref, wy2_ref, by2_ref,
               wa1g_ref, wa1q_ref, wa1t_ref, ba1_ref, wa2_ref, ba2_ref,
               y_ref, ax_ref):
    f32 = jnp.float32
    oh = (lax.broadcasted_iota(jnp.int32, (N, B), 1)
          == batch_ref[...]).astype(f32)
    h = h_ref[...]
    g = lax.dot_general(oh, h, (((0,), (0,)), ((), ())),
                        preferred_element_type=f32)
    ones = jnp.ones((N, 1), f32)
    cnt = lax.dot_general(oh, ones, (((0,), (0,)), ((), ())),
                          preferred_element_type=f32)
    g = g / jnp.maximum(cnt, 1.0)

    qh = jnp.maximum(
        jnp.dot(q_ref[...], wq1_ref[...], preferred_element_type=f32)
        + bq1_ref[...], 0.0)
    qh = jnp.dot(qh, wq2_ref[...], preferred_element_type=f32) + bq2_ref[...]

    oht = (lax.broadcasted_iota(jnp.int32, (B, TV), 1)
           == tc_ref[...]).astype(f32)
    th = jnp.dot(oht, temb_ref[...], preferred_element_type=f32)

    zy = jnp.maximum(
        jnp.dot(g, wy1g_ref[...], preferred_element_type=f32)
        + jnp.dot(qh, wy1q_ref[...], preferred_element_type=f32)
        + jnp.dot(th, wy1t_ref[...], preferred_element_type=f32)
        + by1_ref[...], 0.0)
    y_ref[...] = jnp.dot(zy, wy2_ref[...], preferred_element_type=f32) \
        + by2_ref[...]

    za = jnp.maximum(
        jnp.dot(g, wa1g_ref[...], preferred_element_type=f32)
        + jnp.dot(qh, wa1q_ref[...], preferred_element_type=f32)
        + jnp.dot(th, wa1t_ref[...], preferred_element_type=f32)
        + ba1_ref[...], 0.0)
    ax_ref[...] = jnp.dot(za, wa2_ref[...], preferred_element_type=f32) \
        + ba2_ref[...]


def _tc_head(h, batch, q, tclass, Temb,
             Wq1, bq1, Wq2, bq2, Wy1, by1, Wy2, by2, Wa1, ba1, Wa2, ba2):
    y, ax = pl.pallas_call(
        _head_body,
        out_shape=(
            jax.ShapeDtypeStruct((B, 1), jnp.float32),
            jax.ShapeDtypeStruct((B, 6), jnp.float32),
        ),
    )(h, batch.reshape(N, 1), q, tclass.reshape(B, 1), Temb,
      Wq1, bq1.reshape(1, H), Wq2, bq2.reshape(1, H),
      Wy1[:H], Wy1[H:2 * H], Wy1[2 * H:], by1.reshape(1, H),
      Wy2, by2.reshape(1, 1),
      Wa1[:H], Wa1[H:2 * H], Wa1[2 * H:], ba1.reshape(1, H),
      Wa2, ba2.reshape(1, 6))
    return y.reshape(-1), ax


def kernel(x, edge_index, edge_attr, q, tclass, batch,
           Wx, bx, We, be, W1, b1, W2, b2, Wq1, bq1, Wq2, bq2, Temb,
           Wy1, by1, Wy2, by2, Wa1, ba1, Wa2, ba2):
    src = edge_index[0]
    dst = edge_index[1]
    h = _tc_h0(x, Wx, bx)
    for l in range(L):
        ep = _tc_ep(edge_attr, We[l], be[l])
        aggr2 = _sc_aggr(h, ep, src, dst)
        h = _tc_mlp(h, aggr2, W1[l], b1[l], W2[l], b2[l])
    return _tc_head(h, batch, q, tclass, Temb,
                    Wq1, bq1, Wq2, bq2, Wy1, by1, Wy2, by2,
                    Wa1, ba1, Wa2, ba2)
